# baseline (speedup 1.0000x reference)
import functools

import jax
import jax.numpy as jnp
from jax import lax
from jax.experimental import pallas as pl
from jax.experimental.pallas import tpu as pltpu
from jax.experimental.pallas import tpu_sc as plsc

VOCAB = 1000000
SEQ_LEN = 200
EMBED_DIM = 64
IMG_FEAT = 2048
OUT_DIM = 1000

_NC = 2
_NS = 16
_NW = _NC * _NS

_ROW_BLK = 200
_GRID = OUT_DIM // _ROW_BLK


_ROWS_PER_W = 7
_SLOT_W = 8


_LANE = 128
_STAGE_W = _ROWS_PER_W * _LANE


def _sc_gather_sum_body(idx_hbm, tablet_hbm, out_hbm, idx_v, cols_v, acc_v,
                        sem):
    wid = lax.axis_index("s") * _NC + lax.axis_index("c")
    base = wid * _SLOT_W
    pltpu.sync_copy(idx_hbm.at[pl.ds(base, _SLOT_W)],
                    idx_v.at[pl.ds(0, _SLOT_W)])
    iv = idx_v[...]
    copies = []
    for k in range(_ROWS_PER_W):
        off = pl.multiple_of((iv[k] // _LANE) * _LANE, _LANE)
        copies.append(
            pltpu.async_copy(tablet_hbm.at[:, pl.ds(off, _LANE)],
                             cols_v.at[:, pl.ds(k * _LANE, _LANE)], sem))
    for c in copies:
        c.wait()
    rows16 = lax.iota(jnp.int32, 16)
    for d in range(EMBED_DIM // 16):
        s = jnp.zeros((16,), jnp.float32)
        for k in range(_ROWS_PER_W):
            flag = (wid * _ROWS_PER_W + k < SEQ_LEN).astype(jnp.float32)
            col = jnp.full((16,), k * _LANE, jnp.int32) + (iv[k] % _LANE)
            s = s + plsc.load_gather(cols_v, [rows16 + d * 16, col]) * flag
        acc_v[pl.ds(d * 16, 16)] = s
    pltpu.sync_copy(acc_v, out_hbm.at[wid])


@functools.cache
def _sc_gather_sum():
    return pl.kernel(
        _sc_gather_sum_body,
        out_type=jax.ShapeDtypeStruct((_NW, EMBED_DIM), jnp.float32),
        mesh=plsc.VectorSubcoreMesh(core_axis_name="c", subcore_axis_name="s"),
        scratch_types=[
            pltpu.VMEM((16,), jnp.int32),
            pltpu.VMEM((EMBED_DIM, _STAGE_W), jnp.float32),
            pltpu.VMEM((EMBED_DIM,), jnp.float32),
            pltpu.SemaphoreType.DMA,
        ],
        compiler_params=pltpu.CompilerParams(needs_layout_passes=False),
    )


def _tc_matvec_body(img_hbm, w_ref, out_ref, buf, sems):
    def start(j):
        return pltpu.make_async_copy(
            img_hbm.at[pl.ds(j * _ROW_BLK, _ROW_BLK), 0, :],
            buf.at[j], sems.at[j])

    for j in range(_GRID):
        start(j).start()
    wi = w_ref[:, EMBED_DIM:]
    for j in range(_GRID):
        start(j).wait()
        blk = lax.dot_general(
            wi, buf[j],
            (((1,), (1,)), ((), ())),
            preferred_element_type=jnp.float32,
        )
        out_ref[:, j * _ROW_BLK:(j + 1) * _ROW_BLK] = blk


_tc_matvec = pl.pallas_call(
    _tc_matvec_body,
    in_specs=[
        pl.BlockSpec(memory_space=pltpu.MemorySpace.HBM),
        pl.BlockSpec((1, EMBED_DIM + IMG_FEAT), lambda: (0, 0)),
    ],
    out_specs=pl.BlockSpec((1, OUT_DIM), lambda: (0, 0)),
    out_shape=jax.ShapeDtypeStruct((1, OUT_DIM), jnp.float32),
    scratch_shapes=[
        pltpu.VMEM((_GRID, _ROW_BLK, IMG_FEAT), jnp.float32),
        pltpu.SemaphoreType.DMA((_GRID,)),
    ],
)


def _tc_combine_body(score_ref, w_ref, part_ref, b_ref, out_ref):
    se = jnp.sum(part_ref[...], axis=0, keepdims=True)
    t = jnp.sum(se * w_ref[:, :EMBED_DIM]) + b_ref[0, 0]
    s = score_ref[...] + t
    m = jnp.max(s)
    e = jnp.exp(s - m)
    out_ref[...] = e / jnp.sum(e)


_tc_combine = pl.pallas_call(
    _tc_combine_body,
    in_specs=[
        pl.BlockSpec((1, OUT_DIM), lambda: (0, 0)),
        pl.BlockSpec((1, EMBED_DIM + IMG_FEAT), lambda: (0, 0)),
        pl.BlockSpec((_NW, EMBED_DIM), lambda: (0, 0)),
        pl.BlockSpec((1, 1), lambda: (0, 0)),
    ],
    out_specs=pl.BlockSpec((1, OUT_DIM), lambda: (0, 0)),
    out_shape=jax.ShapeDtypeStruct((1, OUT_DIM), jnp.float32),
)


def kernel(text_input, image_input, emb_table, W, b):
    idx = text_input.reshape(SEQ_LEN).astype(jnp.int32)
    idx7 = jnp.pad(idx, (0, _NW * _ROWS_PER_W - SEQ_LEN)).reshape(
        _NW, _ROWS_PER_W)
    idx_flat = jnp.pad(idx7, ((0, 0), (0, _SLOT_W - _ROWS_PER_W))).reshape(
        _NW * _SLOT_W)
    partials = _sc_gather_sum()(idx_flat, emb_table.T)
    scores = _tc_matvec(image_input, W)
    return _tc_combine(scores, W, partials, b.reshape(1, 1))


# speedup vs baseline: 1.0662x; 1.0662x over previous
import functools

import jax
import jax.numpy as jnp
from jax import lax
from jax.experimental import pallas as pl
from jax.experimental.pallas import tpu as pltpu
from jax.experimental.pallas import tpu_sc as plsc

VOCAB = 1000000
SEQ_LEN = 200
EMBED_DIM = 64
IMG_FEAT = 2048
OUT_DIM = 1000

_NC = 2
_NS = 16
_NW = _NC * _NS

_ROW_BLK = 200
_GRID = OUT_DIM // _ROW_BLK


_ROWS_PER_W = 8
_ACTIVE_W = SEQ_LEN // _ROWS_PER_W


_LANE = 128
_STAGE_W = _ROWS_PER_W * _LANE


def _sc_gather_sum_body(idx_hbm, tablet_hbm, out_hbm, idx_v, cols_v, acc_v,
                        sem):
    wid = lax.axis_index("s") * _NC + lax.axis_index("c")

    @pl.when(wid < _ACTIVE_W)
    def _():
        base = wid * _ROWS_PER_W
        pltpu.sync_copy(idx_hbm.at[pl.ds(base, _ROWS_PER_W)],
                        idx_v.at[pl.ds(0, _ROWS_PER_W)])
        iv = idx_v[...]
        copies = []
        for k in range(_ROWS_PER_W):
            off = pl.multiple_of((iv[k] // _LANE) * _LANE, _LANE)
            copies.append(
                pltpu.async_copy(tablet_hbm.at[:, pl.ds(off, _LANE)],
                                 cols_v.at[:, pl.ds(k * _LANE, _LANE)], sem))
        for c in copies:
            c.wait()
        rows16 = lax.iota(jnp.int32, 16)
        for d in range(EMBED_DIM // 16):
            s = jnp.zeros((16,), jnp.float32)
            for k in range(_ROWS_PER_W):
                col = jnp.full((16,), k * _LANE, jnp.int32) + (iv[k] % _LANE)
                s = s + plsc.load_gather(cols_v, [rows16 + d * 16, col])
            acc_v[pl.ds(d * 16, 16)] = s
        pltpu.sync_copy(acc_v, out_hbm.at[wid])

    @pl.when(wid >= _ACTIVE_W)
    def _():
        for d in range(EMBED_DIM // 16):
            acc_v[pl.ds(d * 16, 16)] = jnp.zeros((16,), jnp.float32)
        pltpu.sync_copy(acc_v, out_hbm.at[wid])


@functools.cache
def _sc_gather_sum():
    return pl.kernel(
        _sc_gather_sum_body,
        out_type=jax.ShapeDtypeStruct((_NW, EMBED_DIM), jnp.float32),
        mesh=plsc.VectorSubcoreMesh(core_axis_name="c", subcore_axis_name="s"),
        scratch_types=[
            pltpu.VMEM((16,), jnp.int32),
            pltpu.VMEM((EMBED_DIM, _STAGE_W), jnp.float32),
            pltpu.VMEM((EMBED_DIM,), jnp.float32),
            pltpu.SemaphoreType.DMA,
        ],
        compiler_params=pltpu.CompilerParams(needs_layout_passes=False),
    )


def _tc_matvec_body(img_hbm, w_ref, out_ref, buf, sems):
    def start(j):
        return pltpu.make_async_copy(
            img_hbm.at[pl.ds(j * _ROW_BLK, _ROW_BLK), 0, :],
            buf.at[j], sems.at[j])

    for j in range(_GRID):
        start(j).start()
    wi = w_ref[:, EMBED_DIM:]
    for j in range(_GRID):
        start(j).wait()
        blk = lax.dot_general(
            wi, buf[j],
            (((1,), (1,)), ((), ())),
            preferred_element_type=jnp.float32,
        )
        out_ref[:, j * _ROW_BLK:(j + 1) * _ROW_BLK] = blk


_tc_matvec = pl.pallas_call(
    _tc_matvec_body,
    in_specs=[
        pl.BlockSpec(memory_space=pltpu.MemorySpace.HBM),
        pl.BlockSpec((1, EMBED_DIM + IMG_FEAT), lambda: (0, 0)),
    ],
    out_specs=pl.BlockSpec((1, OUT_DIM), lambda: (0, 0)),
    out_shape=jax.ShapeDtypeStruct((1, OUT_DIM), jnp.float32),
    scratch_shapes=[
        pltpu.VMEM((_GRID, _ROW_BLK, IMG_FEAT), jnp.float32),
        pltpu.SemaphoreType.DMA((_GRID,)),
    ],
)


def _tc_combine_body(score_ref, w_ref, part_ref, b_ref, out_ref):
    se = jnp.sum(part_ref[...], axis=0, keepdims=True)
    t = jnp.sum(se * w_ref[:, :EMBED_DIM]) + b_ref[0, 0]
    s = score_ref[...] + t
    m = jnp.max(s)
    e = jnp.exp(s - m)
    out_ref[...] = e / jnp.sum(e)


_tc_combine = pl.pallas_call(
    _tc_combine_body,
    in_specs=[
        pl.BlockSpec((1, OUT_DIM), lambda: (0, 0)),
        pl.BlockSpec((1, EMBED_DIM + IMG_FEAT), lambda: (0, 0)),
        pl.BlockSpec((_NW, EMBED_DIM), lambda: (0, 0)),
        pl.BlockSpec((1, 1), lambda: (0, 0)),
    ],
    out_specs=pl.BlockSpec((1, OUT_DIM), lambda: (0, 0)),
    out_shape=jax.ShapeDtypeStruct((1, OUT_DIM), jnp.float32),
)


def kernel(text_input, image_input, emb_table, W, b):
    idx = text_input.reshape(SEQ_LEN).astype(jnp.int32)
    partials = _sc_gather_sum()(idx, emb_table.T)
    scores = _tc_matvec(image_input, W)
    return _tc_combine(scores, W, partials, b.reshape(1, 1))
